# Initial kernel scaffold; baseline (speedup 1.0000x reference)
import jax
import jax.numpy as jnp
from jax.experimental import pallas as pl


def kernel(params, seq, pos, neg, root, item_ids, tax_ids, i2t_src, i2t_dst, t2t_src, t2t_dst, batch_num_tax):
    raise NotImplementedError("write your pallas kernel here")


# speedup vs baseline: 9.6224x; 9.6224x over previous
import functools

import jax
import jax.numpy as jnp
from jax import lax
from jax.experimental import pallas as pl
from jax.experimental.pallas import tpu as pltpu

B = 1024
L = 200
HID = 128
GIP = 2
SCB = 256
FCB_HEADS = 4
FCB = 256
GH = 4
GD = 32
ITEM_NUM = 100000
TAX_NUM = 1000
FIRST = 10
PER_TAX = 20
N_ITEM = B * L
N_TAX = B * PER_TAX
E_I2T = N_ITEM
E_T2T = 2 * N_TAX
TW = 144


def _gelu(x):
    return x * 0.5 * (1.0 + lax.erf(x * (2.0 ** -0.5)))


def _lnT(xT, s, b, eps=1e-8):
    m = jnp.mean(xT, axis=0, keepdims=True)
    v = jnp.mean((xT - m) ** 2, axis=0, keepdims=True)
    return (xT - m) / jnp.sqrt(v + eps) * s[:, None] + b[:, None]


TOWER_BB = 8


def _tower_body(v_ref, *refs):
    out_ref = refs[-1]
    wv = refs[-2][...]
    for bb in range(TOWER_BB):
        vT = v_ref[bb].T
        for blk in range(GIP):
            (ln1s, ln1b, w1, w2, ln2s, ln2b, W1bd, W2bd, w3) = refs[blk * 9:(blk + 1) * 9]
            nVT = _lnT(vT, ln1s[...], ln1b[...])
            t1 = jnp.dot(nVT, w1[...], preferred_element_type=jnp.float32)
            scbT = jnp.dot(_gelu(t1), w2[...], preferred_element_type=jnp.float32)
            vsT = nVT + scbT
            nVsT = _lnT(vsT, ln2s[...], ln2b[...])
            c1 = jnp.dot(W1bd[...].T, nVsT, preferred_element_type=jnp.float32)
            c2 = jnp.dot(W2bd[...].T, _gelu(c1), preferred_element_type=jnp.float32)
            vT = nVsT + jnp.dot(w3[...].T, c2, preferred_element_type=jnp.float32)
        s = jnp.sum(vT * wv, axis=0, keepdims=True)
        s = s - jnp.max(s, axis=1, keepdims=True)
        e = jnp.exp(s)
        alpha = e / jnp.sum(e, axis=1, keepdims=True)
        out_ref[bb, :] = jnp.sum(vT * alpha, axis=1)


def _seq_tower(V, blocks, wv):
    wrefs = []
    for blk in blocks:
        W1bd = jnp.zeros((HID, FCB_HEADS * FCB), jnp.float32)
        W2bd = jnp.zeros((FCB_HEADS * FCB, HID), jnp.float32)
        hd = HID // FCB_HEADS
        for h in range(FCB_HEADS):
            W1bd = W1bd.at[h * hd:(h + 1) * hd, h * FCB:(h + 1) * FCB].set(blk['fcb_w1'])
            W2bd = W2bd.at[h * FCB:(h + 1) * FCB, h * hd:(h + 1) * hd].set(blk['fcb_w2'])
        wrefs += [blk['scb_ln_s'], blk['scb_ln_b'], blk['scb_w1'], blk['scb_w2'],
                  blk['fcb_ln_s'], blk['fcb_ln_b'], W1bd, W2bd, blk['fcb_w3']]
    wrefs.append(wv)
    n_w = len(wrefs)
    in_specs = [pl.BlockSpec((TOWER_BB, L, HID), lambda i: (i, 0, 0))]
    in_specs += [pl.BlockSpec(w.shape, lambda i, nd=w.ndim: (0,) * nd) for w in wrefs]
    return pl.pallas_call(
        _tower_body,
        grid=(B // TOWER_BB,),
        in_specs=in_specs,
        out_specs=pl.BlockSpec((TOWER_BB, HID), lambda i: (i, 0)),
        out_shape=jax.ShapeDtypeStruct((B, HID), jnp.float32),
    )(V, *wrefs)


def _mk_src_tab_body(x_ref, w_ref, al_ref, out_ref):
    x = x_ref[...]
    h = jnp.dot(x, w_ref[...], preferred_element_type=jnp.float32)
    hr = h.reshape(h.shape[0], GH, GD)
    el4 = jnp.sum(hr * al_ref[...][None], axis=-1)
    ones = jnp.ones((h.shape[0], GH), jnp.float32)
    pad = jnp.zeros((h.shape[0], TW - HID - 2 * GH), jnp.float32)
    out_ref[...] = jnp.concatenate([h, ones, el4, pad], axis=1)


def _mk_er_tab_body(x_ref, w_ref, ar_ref, out_ref):
    x = x_ref[...]
    h = jnp.dot(x, w_ref[...], preferred_element_type=jnp.float32)
    hr = h.reshape(h.shape[0], GH, GD)
    er4 = jnp.sum(hr * ar_ref[...][None], axis=-1)
    pad = jnp.zeros((h.shape[0], 12), jnp.float32)
    out_ref[...] = jnp.concatenate([er4, pad], axis=1)


def _mk_src_tab(x, W, al, chunk):
    n = x.shape[0]
    return pl.pallas_call(
        _mk_src_tab_body,
        grid=(n // chunk,),
        in_specs=[pl.BlockSpec((chunk, HID), lambda i: (i, 0)),
                  pl.BlockSpec(W.shape, lambda i: (0, 0)),
                  pl.BlockSpec(al.shape, lambda i: (0, 0))],
        out_specs=pl.BlockSpec((chunk, TW), lambda i: (i, 0)),
        out_shape=jax.ShapeDtypeStruct((n, TW), jnp.float32),
    )(x, W, al)


def _mk_er_tab(x, W, ar, chunk):
    n = x.shape[0]
    return pl.pallas_call(
        _mk_er_tab_body,
        grid=(n // chunk,),
        in_specs=[pl.BlockSpec((chunk, HID), lambda i: (i, 0)),
                  pl.BlockSpec(W.shape, lambda i: (0, 0)),
                  pl.BlockSpec(ar.shape, lambda i: (0, 0))],
        out_specs=pl.BlockSpec((chunk, 16), lambda i: (i, 0)),
        out_shape=jax.ShapeDtypeStruct((n, 16), jnp.float32),
    )(x, W, ar)


def _gat_edge_xla(src_tab, er_tab, src, dst, n_dst):
    rows = src_tab[src]
    el = rows[:, HID + GH:HID + 2 * GH]
    er = er_tab[dst][:, :GH]
    z = el + er
    e = jnp.maximum(z, 0.2 * z)
    ex = jnp.exp(e)
    mult_main = jnp.repeat(ex, GD, axis=1)
    mult_tail = jnp.tile(ex, (1, (TW - HID) // GH))
    vals = rows * jnp.concatenate([mult_main, mult_tail], axis=1)
    return jax.ops.segment_sum(vals, dst, num_segments=n_dst)


def _norm_body(acc1_ref, acc2_ref, b1_ref, b2_ref, exp_ref, out_ref):
    expm = exp_ref[...]
    def one(acc_ref, b_ref):
        acc = acc_ref[...]
        den = acc[:, HID:HID + GH]
        rec = 1.0 / (den + 1e-9)
        recx = jnp.dot(rec, expm, preferred_element_type=jnp.float32)
        return acc[:, :HID] * recx + b_ref[...][None]
    out_ref[...] = one(acc1_ref, b1_ref) + one(acc2_ref, b2_ref)


def _norm_combine(acc1, acc2, b1, b2, chunk=2048):
    n = acc1.shape[0]
    expm = jnp.zeros((GH, HID), jnp.float32)
    for h in range(GH):
        expm = expm.at[h, h * GD:(h + 1) * GD].set(1.0)
    return pl.pallas_call(
        _norm_body,
        grid=(n // chunk,),
        in_specs=[pl.BlockSpec((chunk, TW), lambda i: (i, 0)),
                  pl.BlockSpec((chunk, TW), lambda i: (i, 0)),
                  pl.BlockSpec((HID,), lambda i: (0,)),
                  pl.BlockSpec((HID,), lambda i: (0,)),
                  pl.BlockSpec((GH, HID), lambda i: (0, 0))],
        out_specs=pl.BlockSpec((chunk, HID), lambda i: (i, 0)),
        out_shape=jax.ShapeDtypeStruct((n, HID), jnp.float32),
    )(acc1, acc2, b1, b2, expm)


def _final_body(local_ref, gint_ref, pos_ref, neg_ref, out_ref):
    local = local_ref[...]
    g = gint_ref[...]
    mul = jnp.sum(local * g[:, None, :], axis=-1)
    masked = jnp.where(mul != 0, mul, -jnp.inf)
    m = jnp.max(masked, axis=-1, keepdims=True)
    e = jnp.exp(masked - m)
    w = e / jnp.sum(e, axis=-1, keepdims=True)
    intention = g + jnp.sum(w[:, :, None] * local, axis=1)
    out_ref[0, :, :] = jnp.stack([
        jnp.sum(intention * pos_ref[...], axis=-1),
        jnp.sum(intention * neg_ref[...], axis=-1)], axis=0)


def _final(local, g_int, pos_e, neg_e, chunk=128):
    out = pl.pallas_call(
        _final_body,
        grid=(B // chunk,),
        in_specs=[pl.BlockSpec((chunk, FIRST, HID), lambda i: (i, 0, 0)),
                  pl.BlockSpec((chunk, HID), lambda i: (i, 0)),
                  pl.BlockSpec((chunk, HID), lambda i: (i, 0)),
                  pl.BlockSpec((chunk, HID), lambda i: (i, 0))],
        out_specs=pl.BlockSpec((1, 2, chunk), lambda i: (i, 0, 0)),
        out_shape=jax.ShapeDtypeStruct((B // chunk, 2, chunk), jnp.float32),
    )(local, g_int, pos_e, neg_e)
    out = jnp.swapaxes(out, 0, 1).reshape(2, B)
    return out[0], out[1]


def _gather_rows_xla(table, idx):
    return table[idx]


def kernel(params, seq, pos, neg, root, item_ids, tax_ids,
           i2t_src, i2t_dst, t2t_src, t2t_dst, batch_num_tax):
    item_embed = params['item_embed']
    tax_embed = params['tax_embed']

    V = _gather_rows_xla(item_embed, seq.reshape(-1)).reshape(B, L, HID)
    g_int = _seq_tower(V, params['blocks'], params['wv'])

    item_h = _gather_rows_xla(item_embed, item_ids)
    tax_h = _gather_rows_xla(tax_embed, tax_ids)
    for lyr in params['gnn']:
        ali = lyr['ali']
        art_i = lyr['ari']
        src_tab_i = _mk_src_tab(item_h, lyr['Wi'], ali, 2048)
        er_tab_i = _mk_er_tab(tax_h, lyr['Wi'], art_i, 2048)
        src_tab_t = _mk_src_tab(tax_h, lyr['Wt'], lyr['alt'], 2048)
        er_tab_t = _mk_er_tab(tax_h, lyr['Wt'], lyr['art'], 2048)
        acc_i = _gat_edge_xla(src_tab_i, er_tab_i, i2t_src, i2t_dst, N_TAX)
        acc_t = _gat_edge_xla(src_tab_t, er_tab_t, t2t_src, t2t_dst, N_TAX)
        tax_h = _norm_combine(acc_i, acc_t, lyr['bi'], lyr['bt'])

    tmp = jnp.roll(jnp.cumsum(batch_num_tax), 1).at[0].set(0)
    root_idx = (root + tmp[:, None]).reshape(-1)
    local = _gather_rows_xla(tax_h, root_idx).reshape(B, FIRST, HID)
    valid = (root != -1)
    local = jnp.where(valid[:, :, None], local, 0.0)
    pn = _gather_rows_xla(item_embed, jnp.concatenate([pos, neg]))
    pos_e, neg_e = pn[:B], pn[B:]
    return _final(local, g_int, pos_e, neg_e)


# speedup vs baseline: 11.1688x; 1.1607x over previous
import functools

import jax
import jax.numpy as jnp
from jax import lax
from jax.experimental import pallas as pl
from jax.experimental.pallas import tpu as pltpu
from jax.experimental.pallas import tpu_sc as plsc

NW = 32

B = 1024
L = 200
HID = 128
GIP = 2
SCB = 256
FCB_HEADS = 4
FCB = 256
GH = 4
GD = 32
ITEM_NUM = 100000
TAX_NUM = 1000
FIRST = 10
PER_TAX = 20
N_ITEM = B * L
N_TAX = B * PER_TAX
E_I2T = N_ITEM
E_T2T = 2 * N_TAX
TW = 144


def _gelu(x):
    return x * 0.5 * (1.0 + lax.erf(x * (2.0 ** -0.5)))


def _lnT(xT, s, b, eps=1e-8):
    m = jnp.mean(xT, axis=0, keepdims=True)
    v = jnp.mean((xT - m) ** 2, axis=0, keepdims=True)
    return (xT - m) / jnp.sqrt(v + eps) * s[:, None] + b[:, None]


TOWER_BB = 8


def _tower_body(v_ref, *refs):
    out_ref = refs[-1]
    wv = refs[-2][...]
    for bb in range(TOWER_BB):
        vT = v_ref[bb].T
        for blk in range(GIP):
            (ln1s, ln1b, w1, w2, ln2s, ln2b, W1bd, W2bd, w3) = refs[blk * 9:(blk + 1) * 9]
            nVT = _lnT(vT, ln1s[...], ln1b[...])
            t1 = jnp.dot(nVT, w1[...], preferred_element_type=jnp.float32)
            scbT = jnp.dot(_gelu(t1), w2[...], preferred_element_type=jnp.float32)
            vsT = nVT + scbT
            nVsT = _lnT(vsT, ln2s[...], ln2b[...])
            c1 = jnp.dot(W1bd[...].T, nVsT, preferred_element_type=jnp.float32)
            c2 = jnp.dot(W2bd[...].T, _gelu(c1), preferred_element_type=jnp.float32)
            vT = nVsT + jnp.dot(w3[...].T, c2, preferred_element_type=jnp.float32)
        s = jnp.sum(vT * wv, axis=0, keepdims=True)
        s = s - jnp.max(s, axis=1, keepdims=True)
        e = jnp.exp(s)
        alpha = e / jnp.sum(e, axis=1, keepdims=True)
        out_ref[bb, :] = jnp.sum(vT * alpha, axis=1)


def _seq_tower(V, blocks, wv):
    wrefs = []
    for blk in blocks:
        W1bd = jnp.zeros((HID, FCB_HEADS * FCB), jnp.float32)
        W2bd = jnp.zeros((FCB_HEADS * FCB, HID), jnp.float32)
        hd = HID // FCB_HEADS
        for h in range(FCB_HEADS):
            W1bd = W1bd.at[h * hd:(h + 1) * hd, h * FCB:(h + 1) * FCB].set(blk['fcb_w1'])
            W2bd = W2bd.at[h * FCB:(h + 1) * FCB, h * hd:(h + 1) * hd].set(blk['fcb_w2'])
        wrefs += [blk['scb_ln_s'], blk['scb_ln_b'], blk['scb_w1'], blk['scb_w2'],
                  blk['fcb_ln_s'], blk['fcb_ln_b'], W1bd, W2bd, blk['fcb_w3']]
    wrefs.append(wv)
    n_w = len(wrefs)
    in_specs = [pl.BlockSpec((TOWER_BB, L, HID), lambda i: (i, 0, 0))]
    in_specs += [pl.BlockSpec(w.shape, lambda i, nd=w.ndim: (0,) * nd) for w in wrefs]
    return pl.pallas_call(
        _tower_body,
        grid=(B // TOWER_BB,),
        in_specs=in_specs,
        out_specs=pl.BlockSpec((TOWER_BB, HID), lambda i: (i, 0)),
        out_shape=jax.ShapeDtypeStruct((B, HID), jnp.float32),
    )(V, *wrefs)


def _mk_src_tab_body(x_ref, w_ref, al_ref, out_ref):
    x = x_ref[...]
    h = jnp.dot(x, w_ref[...], preferred_element_type=jnp.float32)
    hr = h.reshape(h.shape[0], GH, GD)
    el4 = jnp.sum(hr * al_ref[...][None], axis=-1)
    ones = jnp.ones((h.shape[0], GH), jnp.float32)
    pad = jnp.zeros((h.shape[0], TW - HID - 2 * GH), jnp.float32)
    out_ref[...] = jnp.concatenate([h, ones, el4, pad], axis=1)


def _mk_er_tab_body(x_ref, w_ref, ar_ref, out_ref):
    x = x_ref[...]
    h = jnp.dot(x, w_ref[...], preferred_element_type=jnp.float32)
    hr = h.reshape(h.shape[0], GH, GD)
    er4 = jnp.sum(hr * ar_ref[...][None], axis=-1)
    pad = jnp.zeros((h.shape[0], 12), jnp.float32)
    out_ref[...] = jnp.concatenate([er4, pad], axis=1)


def _mk_src_tab(x, W, al, chunk):
    n = x.shape[0]
    return pl.pallas_call(
        _mk_src_tab_body,
        grid=(n // chunk,),
        in_specs=[pl.BlockSpec((chunk, HID), lambda i: (i, 0)),
                  pl.BlockSpec(W.shape, lambda i: (0, 0)),
                  pl.BlockSpec(al.shape, lambda i: (0, 0))],
        out_specs=pl.BlockSpec((chunk, TW), lambda i: (i, 0)),
        out_shape=jax.ShapeDtypeStruct((n, TW), jnp.float32),
    )(x, W, al)


def _mk_er_tab(x, W, ar, chunk):
    n = x.shape[0]
    return pl.pallas_call(
        _mk_er_tab_body,
        grid=(n // chunk,),
        in_specs=[pl.BlockSpec((chunk, HID), lambda i: (i, 0)),
                  pl.BlockSpec(W.shape, lambda i: (0, 0)),
                  pl.BlockSpec(ar.shape, lambda i: (0, 0))],
        out_specs=pl.BlockSpec((chunk, 16), lambda i: (i, 0)),
        out_shape=jax.ShapeDtypeStruct((n, 16), jnp.float32),
    )(x, W, ar)


def _gat_edge_xla(src_tab, er_tab, src, dst, n_dst):
    rows = src_tab[src]
    el = rows[:, HID + GH:HID + 2 * GH]
    er = er_tab[dst][:, :GH]
    z = el + er
    e = jnp.maximum(z, 0.2 * z)
    ex = jnp.exp(e)
    mult_main = jnp.repeat(ex, GD, axis=1)
    mult_tail = jnp.tile(ex, (1, (TW - HID) // GH))
    vals = rows * jnp.concatenate([mult_main, mult_tail], axis=1)
    return jax.ops.segment_sum(vals, dst, num_segments=n_dst)


def _norm_body(acc1_ref, acc2_ref, b1_ref, b2_ref, exp_ref, out_ref):
    expm = exp_ref[...]
    def one(acc_ref, b_ref):
        acc = acc_ref[...]
        den = acc[:, HID:HID + GH]
        rec = 1.0 / (den + 1e-9)
        recx = jnp.dot(rec, expm, preferred_element_type=jnp.float32)
        return acc[:, :HID] * recx + b_ref[...][None]
    out_ref[...] = one(acc1_ref, b1_ref) + one(acc2_ref, b2_ref)


def _norm_combine(acc1, acc2, b1, b2, chunk=2048):
    n = acc1.shape[0]
    expm = jnp.zeros((GH, HID), jnp.float32)
    for h in range(GH):
        expm = expm.at[h, h * GD:(h + 1) * GD].set(1.0)
    return pl.pallas_call(
        _norm_body,
        grid=(n // chunk,),
        in_specs=[pl.BlockSpec((chunk, TW), lambda i: (i, 0)),
                  pl.BlockSpec((chunk, TW), lambda i: (i, 0)),
                  pl.BlockSpec((HID,), lambda i: (0,)),
                  pl.BlockSpec((HID,), lambda i: (0,)),
                  pl.BlockSpec((GH, HID), lambda i: (0, 0))],
        out_specs=pl.BlockSpec((chunk, HID), lambda i: (i, 0)),
        out_shape=jax.ShapeDtypeStruct((n, HID), jnp.float32),
    )(acc1, acc2, b1, b2, expm)


def _final_body(local_ref, gint_ref, pos_ref, neg_ref, out_ref):
    local = local_ref[...]
    g = gint_ref[...]
    mul = jnp.sum(local * g[:, None, :], axis=-1)
    masked = jnp.where(mul != 0, mul, -jnp.inf)
    m = jnp.max(masked, axis=-1, keepdims=True)
    e = jnp.exp(masked - m)
    w = e / jnp.sum(e, axis=-1, keepdims=True)
    intention = g + jnp.sum(w[:, :, None] * local, axis=1)
    out_ref[0, :, :] = jnp.stack([
        jnp.sum(intention * pos_ref[...], axis=-1),
        jnp.sum(intention * neg_ref[...], axis=-1)], axis=0)


def _final(local, g_int, pos_e, neg_e, chunk=128):
    out = pl.pallas_call(
        _final_body,
        grid=(B // chunk,),
        in_specs=[pl.BlockSpec((chunk, FIRST, HID), lambda i: (i, 0, 0)),
                  pl.BlockSpec((chunk, HID), lambda i: (i, 0)),
                  pl.BlockSpec((chunk, HID), lambda i: (i, 0)),
                  pl.BlockSpec((chunk, HID), lambda i: (i, 0))],
        out_specs=pl.BlockSpec((1, 2, chunk), lambda i: (i, 0, 0)),
        out_shape=jax.ShapeDtypeStruct((B // chunk, 2, chunk), jnp.float32),
    )(local, g_int, pos_e, neg_e)
    out = jnp.swapaxes(out, 0, 1).reshape(2, B)
    return out[0], out[1]


@functools.lru_cache(maxsize=None)
def _mk_sc_gather(V, D, N, chunk):
    assert N % (8 * NW) == 0 and (N // NW) % chunk == 0 and chunk % 8 == 0
    b_per_w = N // NW
    n_iter = b_per_w // chunk
    mesh = plsc.VectorSubcoreMesh(core_axis_name="c", subcore_axis_name="s")

    @functools.partial(
        pl.kernel, mesh=mesh,
        out_type=jax.ShapeDtypeStruct((N, D), jnp.float32),
        scratch_types=[
            pltpu.VMEM((chunk,), jnp.int32),
            pltpu.VMEM((chunk, D), jnp.float32),
            pltpu.SemaphoreType.DMA,
        ],
    )
    def k(table_hbm, idx_hbm, out_hbm, idx_v, rows_v, sem):
        wid = lax.axis_index("s") * 2 + lax.axis_index("c")
        base = wid * b_per_w

        def body(i, carry):
            off = base + i * chunk
            pltpu.sync_copy(idx_hbm.at[pl.ds(off, chunk)], idx_v)
            pltpu.async_copy(table_hbm.at[idx_v], rows_v, sem).wait()
            pltpu.sync_copy(rows_v, out_hbm.at[pl.ds(off, chunk)])
            return carry

        lax.fori_loop(0, n_iter, body, 0)

    return k


def _gather_rows(table, idx, chunk=640):
    return _mk_sc_gather(table.shape[0], table.shape[1], idx.shape[0], chunk)(table, idx)


def kernel(params, seq, pos, neg, root, item_ids, tax_ids,
           i2t_src, i2t_dst, t2t_src, t2t_dst, batch_num_tax):
    item_embed = params['item_embed']
    tax_embed = params['tax_embed']

    V = _gather_rows(item_embed, seq.reshape(-1)).reshape(B, L, HID)
    g_int = _seq_tower(V, params['blocks'], params['wv'])

    item_h = _gather_rows(item_embed, item_ids)
    tax_h = _gather_rows(tax_embed, tax_ids)
    for lyr in params['gnn']:
        ali = lyr['ali']
        art_i = lyr['ari']
        src_tab_i = _mk_src_tab(item_h, lyr['Wi'], ali, 2048)
        er_tab_i = _mk_er_tab(tax_h, lyr['Wi'], art_i, 2048)
        src_tab_t = _mk_src_tab(tax_h, lyr['Wt'], lyr['alt'], 2048)
        er_tab_t = _mk_er_tab(tax_h, lyr['Wt'], lyr['art'], 2048)
        acc_i = _gat_edge_xla(src_tab_i, er_tab_i, i2t_src, i2t_dst, N_TAX)
        acc_t = _gat_edge_xla(src_tab_t, er_tab_t, t2t_src, t2t_dst, N_TAX)
        tax_h = _norm_combine(acc_i, acc_t, lyr['bi'], lyr['bt'])

    tmp = jnp.roll(jnp.cumsum(batch_num_tax), 1).at[0].set(0)
    root_idx = (root + tmp[:, None]).reshape(-1)
    local = _gather_rows(tax_h, root_idx, chunk=320).reshape(B, FIRST, HID)
    valid = (root != -1)
    local = jnp.where(valid[:, :, None], local, 0.0)
    pn = _gather_rows(item_embed, jnp.concatenate([pos, neg]), chunk=64)
    pos_e, neg_e = pn[:B], pn[B:]
    return _final(local, g_int, pos_e, neg_e)
